# baseline (speedup 1.0000x reference)
import functools

import jax
import jax.numpy as jnp
from jax import lax
from jax.experimental import pallas as pl
from jax.experimental.pallas import tpu as pltpu
from jax.experimental.pallas import tpu_sc as plsc

N_MASK = 1024
BS = 64
N = 8192
LANES = 16
NVEC = N // LANES
ROWS_PER_W = 2

_LEVELS = ((22, 512, 31), (15, 128, 22), (8, 128, 15), (0, 256, 8))


def _loss_kernel(out_ref, tgt_ref, loss_ref):
    x = out_ref[...]
    t = tgt_ref[...]
    loss_ref[...] = jnp.maximum(x, 0.0) - x * t + jnp.log1p(jnp.exp(-jnp.abs(x)))


def _tc_loss(output, target):
    return pl.pallas_call(
        _loss_kernel,
        out_shape=jax.ShapeDtypeStruct((BS, N), jnp.float32),
    )(output, target)


def _scan_level(tot_v, toff, nch, rank, lane):
    ngroups = (nch + 15) // LANES
    seen = jnp.int32(0)
    found = jnp.int32(0)
    hc_v = jnp.zeros((LANES,), jnp.int32)
    A_v = jnp.zeros((LANES,), jnp.int32)
    tc_v = jnp.zeros((LANES,), jnp.int32)
    for g in range(ngroups - 1, -1, -1):
        mv = (g * LANES + lane) < nch
        cs = jnp.zeros((LANES,), jnp.int32)
        for w in range(LANES):
            gg = plsc.load_gather(
                tot_v, [toff + (g * LANES + lane) * LANES + w], mask=mv)
            cs = cs + jnp.where(mv, gg, 0)
        pre = plsc.cumsum(cs)
        tg = jnp.sum(jnp.where(lane == LANES - 1, pre, 0))
        suffix = (seen + tg) - pre + cs
        m = (suffix >= rank) & mv
        pc = plsc.all_reduce_population_count(m)
        j_v = pc - 1
        hit = (found == 0) & (jnp.sum(jnp.where(lane == 0, pc, 0)) > 0)
        hit_i = hit.astype(jnp.int32)
        sel = jnp.where(lane == j_v, hit_i, 0)
        hc_v = hc_v + sel * (g * LANES + j_v)
        A_v = A_v + sel * (suffix - cs)
        tc_v = tc_v + sel * cs
        found = found + hit_i
        seen = seen + tg
    hc = jnp.sum(hc_v)
    A = jnp.sum(A_v)
    tot_c = jnp.sum(tc_v)

    t = tot_v[pl.ds(toff + hc * LANES, LANES)]
    pre2 = plsc.cumsum(t)
    suffix3 = (A + tot_c) - pre2 + t
    m2 = suffix3 >= rank
    j2 = plsc.all_reduce_population_count(m2) - 1
    sel2 = jnp.where(lane == j2, 1, 0)
    B = jnp.sum(sel2 * (hc * LANES + j2))
    cnt_ge = jnp.sum(sel2 * suffix3)
    MB = jnp.sum(sel2 * t)
    return B, cnt_ge, MB


def _sc_topk_body(loss_hbm, out_hbm, row0_v, row1_v, hist_v, tot_v, out_v,
                  sem0, sem1):
    wid = lax.axis_index("s") * 2 + lax.axis_index("c")
    lane = lax.broadcasted_iota(jnp.int32, (LANES,), 0)
    ones = jnp.ones((LANES,), jnp.int32)
    zz = jnp.zeros((LANES,), jnp.int32)

    row = wid * ROWS_PER_W
    cp0 = pltpu.async_copy(loss_hbm.at[row], row0_v, sem0)
    cp1 = pltpu.async_copy(loss_hbm.at[row + 1], row1_v, sem1)
    cp0.wait()
    cp1.wait()

    rows = (row0_v, row1_v)
    base = [jnp.int32(0), jnp.int32(0)]
    rank = [jnp.int32(N_MASK), jnp.int32(N_MASK)]

    for shift, nbins, ushift in _LEVELS:
        nch = nbins // LANES

        @plsc.parallel_loop(0, 2 * nch * LANES, unroll=8)
        def _(j):
            hist_v[pl.ds(j * LANES, LANES)] = zz

        pref0 = base[0] >> ushift
        pref1 = base[1] >> ushift
        mbits = jnp.int32(nbins - 1)

        @plsc.parallel_loop(0, NVEC, unroll=8)
        def _(j, shift=shift, ushift=ushift, pref0=pref0, pref1=pref1,
              mbits=mbits, nbins=nbins):
            for rv, pref, off in ((row0_v, pref0, 0),
                                  (row1_v, pref1, nbins * LANES)):
                v = rv[pl.ds(j * LANES, LANES)]
                bits = lax.bitcast_convert_type(v, jnp.int32)
                b = (bits >> shift) & mbits
                m = (bits >> ushift) == pref
                plsc.addupdate_scatter(hist_v, [off + lane * nbins + b],
                                       ones, mask=m)

        @plsc.parallel_loop(0, nch, unroll=2)
        def _(c, nbins=nbins):
            for off, toff in ((0, 0), (nbins * LANES, 512)):
                parts = [hist_v[pl.ds(off + l * nbins + c * LANES, LANES)]
                         for l in range(LANES)]
                while len(parts) > 1:
                    parts = [parts[i] + parts[i + 1]
                             for i in range(0, len(parts), 2)]
                tot_v[pl.ds(toff + c * LANES, LANES)] = parts[0]

        for ridx in range(2):
            B, cnt_ge, MB = _scan_level(tot_v, ridx * 512, nch,
                                        rank[ridx], lane)
            rank[ridx] = rank[ridx] - (cnt_ge - MB)
            base[ridx] = base[ridx] | (B << shift)

    kth0, kth1 = base

    @plsc.parallel_loop(0, NVEC, unroll=8,
                        carry=(jnp.zeros((LANES,), jnp.float32),
                               jnp.zeros((LANES,), jnp.int32),
                               jnp.zeros((LANES,), jnp.float32),
                               jnp.zeros((LANES,), jnp.int32)))
    def final_carry(j, carry, kth0=kth0, kth1=kth1):
        sv0, cv0, sv1, cv1 = carry
        v0 = row0_v[pl.ds(j * LANES, LANES)]
        m0 = lax.bitcast_convert_type(v0, jnp.int32) > kth0
        v1 = row1_v[pl.ds(j * LANES, LANES)]
        m1 = lax.bitcast_convert_type(v1, jnp.int32) > kth1
        return (sv0 + jnp.where(m0, v0, 0.0),
                cv0 + jnp.where(m0, jnp.int32(1), jnp.int32(0)),
                sv1 + jnp.where(m1, v1, 0.0),
                cv1 + jnp.where(m1, jnp.int32(1), jnp.int32(0)))

    sv0, cv0, sv1, cv1 = final_carry
    totals = []
    for kth, sv, cv in ((kth0, sv0, cv0), (kth1, sv1, cv1)):
        thr_v = lax.bitcast_convert_type(jnp.full((LANES,), kth, jnp.int32),
                                         jnp.float32)
        n_tie = (jnp.int32(N_MASK) - jnp.sum(cv)).astype(jnp.float32)
        totals.append(jnp.sum(sv) + n_tie * thr_v)

    out_v[...] = jnp.where(lane == 0, totals[0],
                           jnp.where(lane == 1, totals[1], 0.0))
    pltpu.sync_copy(out_v, out_hbm.at[wid])


@functools.partial(
    pl.kernel,
    out_type=jax.ShapeDtypeStruct((BS // ROWS_PER_W, LANES), jnp.float32),
    mesh=plsc.VectorSubcoreMesh(core_axis_name="c", subcore_axis_name="s"),
    compiler_params=pltpu.CompilerParams(needs_layout_passes=False),
    scratch_types=[
        pltpu.VMEM((N,), jnp.float32),
        pltpu.VMEM((N,), jnp.float32),
        pltpu.VMEM((2 * N,), jnp.int32),
        pltpu.VMEM((1024,), jnp.int32),
        pltpu.VMEM((LANES,), jnp.float32),
        pltpu.SemaphoreType.DMA,
        pltpu.SemaphoreType.DMA,
    ],
)
def _sc_topk(loss_hbm, out_hbm, row0_v, row1_v, hist_v, tot_v, out_v,
             sem0, sem1):
    _sc_topk_body(loss_hbm, out_hbm, row0_v, row1_v, hist_v, tot_v, out_v,
                  sem0, sem1)


@jax.jit
def kernel(output, target):
    loss = _tc_loss(output, target)
    row_sums = _sc_topk(loss)
    return (jnp.sum(row_sums) / (BS * N_MASK)).astype(jnp.float32)


# speedup vs baseline: 1.0112x; 1.0112x over previous
import functools

import jax
import jax.numpy as jnp
from jax import lax
from jax.experimental import pallas as pl
from jax.experimental.pallas import tpu as pltpu
from jax.experimental.pallas import tpu_sc as plsc

N_MASK = 1024
BS = 64
N = 8192
LANES = 16
NVEC = N // LANES
ROWS_PER_W = 2

_LEVELS = ((22, 512, 31), (15, 128, 22), (8, 128, 15), (0, 256, 8))


def _loss_kernel(out_ref, tgt_ref, loss_ref):
    x = out_ref[...]
    t = tgt_ref[...]
    loss_ref[...] = jnp.maximum(x, 0.0) - x * t + jnp.log1p(jnp.exp(-jnp.abs(x)))


def _tc_loss(output, target):
    return pl.pallas_call(
        _loss_kernel,
        out_shape=jax.ShapeDtypeStruct((BS, N), jnp.float32),
    )(output, target)


def _scan_level(tot_v, toff, nch, rank, lane):
    ngroups = (nch + 15) // LANES
    seen = jnp.int32(0)
    found = jnp.int32(0)
    hc_v = jnp.zeros((LANES,), jnp.int32)
    A_v = jnp.zeros((LANES,), jnp.int32)
    tc_v = jnp.zeros((LANES,), jnp.int32)
    for g in range(ngroups - 1, -1, -1):
        mv = (g * LANES + lane) < nch
        cs = jnp.zeros((LANES,), jnp.int32)
        for w in range(LANES):
            gg = plsc.load_gather(
                tot_v, [toff + (g * LANES + lane) * LANES + w], mask=mv)
            cs = cs + jnp.where(mv, gg, 0)
        pre = plsc.cumsum(cs)
        tg = jnp.sum(jnp.where(lane == LANES - 1, pre, 0))
        suffix = (seen + tg) - pre + cs
        m = (suffix >= rank) & mv
        pc = plsc.all_reduce_population_count(m)
        j_v = pc - 1
        hit = (found == 0) & (jnp.sum(jnp.where(lane == 0, pc, 0)) > 0)
        hit_i = hit.astype(jnp.int32)
        sel = jnp.where(lane == j_v, hit_i, 0)
        hc_v = hc_v + sel * (g * LANES + j_v)
        A_v = A_v + sel * (suffix - cs)
        tc_v = tc_v + sel * cs
        found = found + hit_i
        seen = seen + tg
    hc = jnp.sum(hc_v)
    A = jnp.sum(A_v)
    tot_c = jnp.sum(tc_v)

    t = tot_v[pl.ds(toff + hc * LANES, LANES)]
    pre2 = plsc.cumsum(t)
    suffix3 = (A + tot_c) - pre2 + t
    m2 = suffix3 >= rank
    j2 = plsc.all_reduce_population_count(m2) - 1
    sel2 = jnp.where(lane == j2, 1, 0)
    B = jnp.sum(sel2 * (hc * LANES + j2))
    cnt_ge = jnp.sum(sel2 * suffix3)
    MB = jnp.sum(sel2 * t)
    return B, cnt_ge, MB


def _sc_topk_body(loss_hbm, out_hbm, row0_v, row1_v, hist_v, tot_v, out_v,
                  sem0, sem1):
    wid = lax.axis_index("s") * 2 + lax.axis_index("c")
    lane = lax.broadcasted_iota(jnp.int32, (LANES,), 0)
    ones = jnp.ones((LANES,), jnp.int32)
    zz = jnp.zeros((LANES,), jnp.int32)

    row = wid * ROWS_PER_W
    cp0 = pltpu.async_copy(loss_hbm.at[row], row0_v, sem0)
    cp1 = pltpu.async_copy(loss_hbm.at[row + 1], row1_v, sem1)
    cp0.wait()
    cp1.wait()

    base = [jnp.int32(0), jnp.int32(0)]
    rank = [jnp.int32(N_MASK), jnp.int32(N_MASK)]

    for shift, nbins, ushift in _LEVELS:
        nch = nbins // LANES

        @plsc.parallel_loop(0, 2 * nch * LANES, unroll=8)
        def _(j):
            hist_v[pl.ds(j * LANES, LANES)] = zz

        pref0 = base[0] >> ushift
        pref1 = base[1] >> ushift
        mbits = jnp.int32(nbins - 1)

        @plsc.parallel_loop(0, NVEC, unroll=4)
        def _(j, shift=shift, ushift=ushift, pref0=pref0, pref1=pref1,
              mbits=mbits, nbins=nbins):
            for rv, pref, off in ((row0_v, pref0, 0),
                                  (row1_v, pref1, nbins * LANES)):
                v = rv[pl.ds(j * LANES, LANES)]
                bits = lax.bitcast_convert_type(v, jnp.int32)
                b = (bits >> shift) & mbits
                m = (bits >> ushift) == pref
                plsc.addupdate_scatter(hist_v, [off + lane * nbins + b],
                                       ones, mask=m)

        @plsc.parallel_loop(0, nch, unroll=2)
        def _(c, nbins=nbins):
            for off, toff in ((0, 0), (nbins * LANES, 512)):
                parts = [hist_v[pl.ds(off + l * nbins + c * LANES, LANES)]
                         for l in range(LANES)]
                while len(parts) > 1:
                    parts = [parts[i] + parts[i + 1]
                             for i in range(0, len(parts), 2)]
                tot_v[pl.ds(toff + c * LANES, LANES)] = parts[0]

        for ridx in range(2):
            B, cnt_ge, MB = _scan_level(tot_v, ridx * 512, nch,
                                        rank[ridx], lane)
            rank[ridx] = rank[ridx] - (cnt_ge - MB)
            base[ridx] = base[ridx] | (B << shift)

    kth0, kth1 = base

    @plsc.parallel_loop(0, NVEC, unroll=4,
                        carry=(jnp.zeros((LANES,), jnp.float32),
                               jnp.zeros((LANES,), jnp.int32),
                               jnp.zeros((LANES,), jnp.float32),
                               jnp.zeros((LANES,), jnp.int32)))
    def final_carry(j, carry, kth0=kth0, kth1=kth1):
        sv0, cv0, sv1, cv1 = carry
        v0 = row0_v[pl.ds(j * LANES, LANES)]
        m0 = lax.bitcast_convert_type(v0, jnp.int32) > kth0
        v1 = row1_v[pl.ds(j * LANES, LANES)]
        m1 = lax.bitcast_convert_type(v1, jnp.int32) > kth1
        return (sv0 + jnp.where(m0, v0, 0.0),
                cv0 + jnp.where(m0, jnp.int32(1), jnp.int32(0)),
                sv1 + jnp.where(m1, v1, 0.0),
                cv1 + jnp.where(m1, jnp.int32(1), jnp.int32(0)))

    sv0, cv0, sv1, cv1 = final_carry
    totals = []
    for kth, sv, cv in ((kth0, sv0, cv0), (kth1, sv1, cv1)):
        thr_v = lax.bitcast_convert_type(jnp.full((LANES,), kth, jnp.int32),
                                         jnp.float32)
        n_tie = (jnp.int32(N_MASK) - jnp.sum(cv)).astype(jnp.float32)
        totals.append(jnp.sum(sv) + n_tie * thr_v)

    out_v[...] = jnp.where(lane == 0, totals[0],
                           jnp.where(lane == 1, totals[1], 0.0))
    pltpu.sync_copy(out_v, out_hbm.at[wid])


@functools.partial(
    pl.kernel,
    out_type=jax.ShapeDtypeStruct((BS // ROWS_PER_W, LANES), jnp.float32),
    mesh=plsc.VectorSubcoreMesh(core_axis_name="c", subcore_axis_name="s"),
    compiler_params=pltpu.CompilerParams(needs_layout_passes=False),
    scratch_types=[
        pltpu.VMEM((N,), jnp.float32),
        pltpu.VMEM((N,), jnp.float32),
        pltpu.VMEM((2 * N,), jnp.int32),
        pltpu.VMEM((1024,), jnp.int32),
        pltpu.VMEM((LANES,), jnp.float32),
        pltpu.SemaphoreType.DMA,
        pltpu.SemaphoreType.DMA,
    ],
)
def _sc_topk(loss_hbm, out_hbm, row0_v, row1_v, hist_v, tot_v, out_v,
             sem0, sem1):
    _sc_topk_body(loss_hbm, out_hbm, row0_v, row1_v, hist_v, tot_v, out_v,
                  sem0, sem1)


@jax.jit
def kernel(output, target):
    loss = _tc_loss(output, target)
    row_sums = _sc_topk(loss)
    return (jnp.sum(row_sums) / (BS * N_MASK)).astype(jnp.float32)
